# baseline (speedup 1.0000x reference)
import jax
import jax.numpy as jnp
from jax.experimental import pallas as pl
from jax.experimental.pallas import tpu as pltpu

_NCLS = 30
_HI = 799.0
_G = 8
_B = 1000


def _clip(v):
    return jnp.clip(v, 1.0, _HI)


def _head_body(gt_ref, lab_ref, p8_ref, props_ref, ov_ref, tm_ref, mask_ref):
    n = props_ref.shape[1]
    px1 = _clip(props_ref[0:1, :])
    py1 = _clip(props_ref[1:2, :])
    px2 = _clip(props_ref[2:3, :])
    py2 = _clip(props_ref[3:4, :])
    area = (px2 - px1 + 1.0) * (py2 - py1 + 1.0)

    ious = []
    for g in range(_G):
        gx1 = _clip(gt_ref[g, 0])
        gy1 = _clip(gt_ref[g, 1])
        gx2 = _clip(gt_ref[g, 2])
        gy2 = _clip(gt_ref[g, 3])
        iw = jnp.maximum(jnp.minimum(px2, gx2) - jnp.maximum(px1, gx1) + 1.0, 0.0)
        ih = jnp.maximum(jnp.minimum(py2, gy2) - jnp.maximum(py1, gy1) + 1.0, 0.0)
        inter = iw * ih
        ag = (gx2 - gx1 + 1.0) * (gy2 - gy1 + 1.0)
        ious.append(inter / (area + ag - inter))

    iota_c = jax.lax.broadcasted_iota(jnp.int32, (_NCLS, 1), 0)
    ov = jnp.zeros((_NCLS, n), jnp.float32)
    for g in range(_G):
        sel = iota_c == lab_ref[g]
        ov = jnp.maximum(ov, jnp.where(sel, ious[g], 0.0))
    ov_ref[...] = ov

    masks = []
    for g in range(_G):
        ol = ious[g]
        for g2 in range(_G):
            if g2 != g:
                same = lab_ref[g] == lab_ref[g2]
                ol = jnp.maximum(ol, jnp.where(same, ious[g2], 0.0))
        masks.append((ol > 0.6).astype(jnp.float32))
    mask_ref[...] = jnp.concatenate(masks, axis=0)

    src_w = px2 - px1
    src_h = py2 - py1
    src_cx = px1 + 0.5 * src_w
    src_cy = py1 + 0.5 * src_h
    for g in range(_G):
        q1 = _clip(p8_ref[g, 0])
        q2 = _clip(p8_ref[g, 1])
        q3 = _clip(p8_ref[g, 2])
        q4 = _clip(p8_ref[g, 3])
        gw = q3 - q1
        gh = q4 - q2
        gcx = q1 + 0.5 * gw
        gcy = q2 + 0.5 * gh
        dcx = (gcx - src_cx) / src_w
        dcy = (gcy - src_cy) / src_h
        dw = jnp.log(gw / src_w)
        dh = jnp.log(gh / src_h)
        t4 = jnp.concatenate([dcx, dcy, dw, dh], axis=0) * masks[g]
        tm_ref[:, g, :] = t4


def _posfeat_body(maskt_ref, x_ref, pf_ref):
    @pl.when(pl.program_id(0) == 0)
    def _():
        pf_ref[...] = jnp.zeros_like(pf_ref)

    pf_ref[...] += jax.lax.dot_general(
        maskt_ref[...],
        x_ref[...],
        (((0,), (0,)), ((), ())),
        preferred_element_type=jnp.float32,
    )


def kernel(x, proposals, gt_bbox, gt_labels):
    n, d = x.shape
    props_t = proposals.T
    p8 = proposals[:_G]

    ov_cn, tm, mask = pl.pallas_call(
        _head_body,
        in_specs=[
            pl.BlockSpec(memory_space=pltpu.SMEM),
            pl.BlockSpec(memory_space=pltpu.SMEM),
            pl.BlockSpec(memory_space=pltpu.SMEM),
            pl.BlockSpec((4, n), lambda: (0, 0)),
        ],
        out_specs=[
            pl.BlockSpec((_NCLS, n), lambda: (0, 0)),
            pl.BlockSpec((4, _G, n), lambda: (0, 0, 0)),
            pl.BlockSpec((_G, n), lambda: (0, 0)),
        ],
        out_shape=[
            jax.ShapeDtypeStruct((_NCLS, n), jnp.float32),
            jax.ShapeDtypeStruct((4, _G, n), jnp.float32),
            jax.ShapeDtypeStruct((_G, n), jnp.float32),
        ],
    )(gt_bbox, gt_labels.astype(jnp.int32), p8, props_t)

    pf = pl.pallas_call(
        _posfeat_body,
        grid=(n // _B,),
        in_specs=[
            pl.BlockSpec((_B, _G), lambda i: (i, 0)),
            pl.BlockSpec((_B, d), lambda i: (i, 0)),
        ],
        out_specs=pl.BlockSpec((_G, d), lambda i: (0, 0)),
        out_shape=jax.ShapeDtypeStruct((_G, d), jnp.float32),
    )(mask.T, x)

    return ov_cn.T, tm.transpose(1, 2, 0), pf
